# baseline (speedup 1.0000x reference)
import jax
import jax.numpy as jnp
from jax.experimental import pallas as pl
from jax.experimental.pallas import tpu as pltpu


def _pos_kernel(col_ref, row_ref, out_ref):
    w, d = col_ref.shape
    h, _ = row_ref.shape
    out_ref[0, :, :, 0:d] = jnp.broadcast_to(col_ref[...][None, :, :], (h, w, d))
    out_ref[0, :, :, d:2 * d] = jnp.broadcast_to(row_ref[...][:, None, :], (h, w, d))


def kernel(tensor_list, row_embed, col_embed):
    b = tensor_list.shape[0]
    h, w = tensor_list.shape[-2], tensor_list.shape[-1]
    d = col_embed.shape[-1]
    out = pl.pallas_call(
        _pos_kernel,
        out_shape=jax.ShapeDtypeStruct((b, h, w, 2 * d), jnp.float32),
        grid=(b,),
        in_specs=[
            pl.BlockSpec((w, d), lambda i: (0, 0)),
            pl.BlockSpec((h, d), lambda i: (0, 0)),
        ],
        out_specs=pl.BlockSpec((1, h, w, 2 * d), lambda i: (i, 0, 0, 0)),
    )(col_embed[:w], row_embed[:h])
    return jnp.transpose(out, (0, 3, 1, 2))


# speedup vs baseline: 1.9772x; 1.9772x over previous
import jax
import jax.numpy as jnp
from jax.experimental import pallas as pl
from jax.experimental.pallas import tpu as pltpu


def _pos_kernel(col_ref, row_ref, out_ref, scr, sem):
    w, d = col_ref.shape
    h, _ = row_ref.shape
    b = out_ref.shape[0]
    scr[:, :, 0:d] = jnp.broadcast_to(col_ref[...][None, :, :], (h, w, d))
    scr[:, :, d:2 * d] = jnp.broadcast_to(row_ref[...][:, None, :], (h, w, d))
    copies = [
        pltpu.make_async_copy(scr, out_ref.at[i], sem.at[i]) for i in range(b)
    ]
    for cp in copies:
        cp.start()
    for cp in copies:
        cp.wait()


def kernel(tensor_list, row_embed, col_embed):
    b = tensor_list.shape[0]
    h, w = tensor_list.shape[-2], tensor_list.shape[-1]
    d = col_embed.shape[-1]
    out = pl.pallas_call(
        _pos_kernel,
        out_shape=jax.ShapeDtypeStruct((b, h, w, 2 * d), jnp.float32),
        grid=(1,),
        in_specs=[
            pl.BlockSpec((w, d), lambda i: (0, 0)),
            pl.BlockSpec((h, d), lambda i: (0, 0)),
        ],
        out_specs=pl.BlockSpec(memory_space=pl.ANY),
        scratch_shapes=[
            pltpu.VMEM((h, w, 2 * d), jnp.float32),
            pltpu.SemaphoreType.DMA((b,)),
        ],
    )(col_embed, row_embed)
    return jnp.transpose(out, (0, 3, 1, 2))


# speedup vs baseline: 1.9874x; 1.0052x over previous
import jax
import jax.numpy as jnp
from jax.experimental import pallas as pl
from jax.experimental.pallas import tpu as pltpu


def _pos_kernel(col_ref, row_ref, out_ref, scr, sem):
    w, d = col_ref.shape
    h, _ = row_ref.shape
    b = out_ref.shape[0]
    scr[:, :, 0:d] = jnp.broadcast_to(col_ref[...][None, :, :], (h, w, d))
    scr[:, :, d:2 * d] = jnp.broadcast_to(row_ref[...][:, None, :], (h, w, d))
    copies = [
        pltpu.make_async_copy(scr, out_ref.at[i], sem.at[i]) for i in range(b)
    ]
    for i, cp in enumerate(copies):
        cp.start(priority=i % 2)
    for cp in copies:
        cp.wait()


def kernel(tensor_list, row_embed, col_embed):
    b = tensor_list.shape[0]
    h, w = tensor_list.shape[-2], tensor_list.shape[-1]
    d = col_embed.shape[-1]
    out = pl.pallas_call(
        _pos_kernel,
        out_shape=jax.ShapeDtypeStruct((b, h, w, 2 * d), jnp.float32),
        grid=(1,),
        in_specs=[
            pl.BlockSpec((w, d), lambda i: (0, 0)),
            pl.BlockSpec((h, d), lambda i: (0, 0)),
        ],
        out_specs=pl.BlockSpec(memory_space=pl.ANY),
        scratch_shapes=[
            pltpu.VMEM((h, w, 2 * d), jnp.float32),
            pltpu.SemaphoreType.DMA((b,)),
        ],
    )(col_embed, row_embed)
    return jnp.transpose(out, (0, 3, 1, 2))
